# Initial kernel scaffold; baseline (speedup 1.0000x reference)
import jax
import jax.numpy as jnp
from jax.experimental import pallas as pl


def kernel(boxes, image_meta, feature_map_p2, feature_map_p3, feature_map_p4, feature_map_p5):
    raise NotImplementedError("write your pallas kernel here")


# speedup vs baseline: 12.8173x; 12.8173x over previous
import functools

import jax
import jax.numpy as jnp
from jax import lax
from jax.experimental import pallas as pl
from jax.experimental.pallas import tpu as pltpu
from jax.experimental.pallas import tpu_sc as plsc

BB, NN = 2, 1000
PH, PW = 7, 7
CH = 256
NBOX = BB * NN
NBOX_PAD = 2048
ROWS = NBOX_PAD * PH * PW
NWORK = 32
ROWS_PER_W = ROWS // NWORK
CHUNK = 16
NCHUNK = ROWS_PER_W // CHUNK

_HL = (256, 128, 64, 32)
_OFF = (0,
        BB * 256 * 256,
        BB * 256 * 256 + BB * 128 * 128,
        BB * 256 * 256 + BB * 128 * 128 + BB * 64 * 64)
_TROWS = _OFF[3] + BB * 32 * 32


def _prep_body(boxes_ref, meta_ref, idx_ref, w_ref):
    b4 = boxes_ref[:]
    y1 = b4[:, 0:1]
    x1 = b4[:, 1:2]
    y2 = b4[:, 2:3]
    x2 = b4[:, 3:4]
    h = y2 - y1
    w = x2 - x1
    ih = meta_ref[0:1, 4:5]
    iw = meta_ref[0:1, 5:6]
    area = ih * iw
    lvl_f = jnp.log(jnp.sqrt(h * w) / (224.0 / jnp.sqrt(area))) / jnp.log(2.0)
    lvl = jnp.minimum(5, jnp.maximum(2, 4 + jnp.round(lvl_f).astype(jnp.int32)))

    hf = jnp.where(lvl == 2, 256.0,
                   jnp.where(lvl == 3, 128.0,
                             jnp.where(lvl == 4, 64.0, 32.0)))
    hi = hf.astype(jnp.int32)
    hw_rows = hi * hi
    off = jnp.where(lvl == 2, _OFF[0],
                    jnp.where(lvl == 3, _OFF[1],
                              jnp.where(lvl == 4, _OFF[2], _OFF[3])))
    bidx = (lax.broadcasted_iota(jnp.int32, (NBOX_PAD, 1), 0) >= NN).astype(jnp.int32)

    q = lax.broadcasted_iota(jnp.int32, (NBOX_PAD, PH * PW * 4), 1)
    pi = q // (PW * 4)
    pj = (q // 4) % PW
    c = q % 4
    fy = pi.astype(jnp.float32)
    fx = pj.astype(jnp.float32)
    ys = y1 * (hf - 1.0) + fy * (h * (hf - 1.0) / 6.0)
    xs = x1 * (hf - 1.0) + fx * (w * (hf - 1.0) / 6.0)
    y0f = jnp.floor(ys)
    x0f = jnp.floor(xs)
    wy = ys - y0f
    wx = xs - x0f
    him1 = hi - 1
    yy0 = jnp.clip(y0f.astype(jnp.int32), 0, him1)
    yy1 = jnp.minimum(yy0 + 1, him1)
    xx0 = jnp.clip(x0f.astype(jnp.int32), 0, him1)
    xx1 = jnp.minimum(xx0 + 1, him1)
    cy = jnp.where(c >= 2, yy1, yy0)
    cx = jnp.where(c % 2 == 1, xx1, xx0)
    wyc = jnp.where(c >= 2, wy, 1.0 - wy)
    wxc = jnp.where(c % 2 == 1, wx, 1.0 - wx)
    flat = off + bidx * hw_rows + cy * hi + cx
    wgt = wyc * wxc
    valid = ((ys >= 0.0) & (ys <= hf - 1.0) & (xs >= 0.0) & (xs <= hf - 1.0))
    wgt = jnp.where(valid, wgt, 0.0)
    idx_ref[:] = flat
    w_ref[:] = wgt


_prep = pl.pallas_call(
    _prep_body,
    out_shape=[
        jax.ShapeDtypeStruct((NBOX_PAD, PH * PW * 4), jnp.int32),
        jax.ShapeDtypeStruct((NBOX_PAD, PH * PW * 4), jnp.float32),
    ],
)


@functools.cache
def _make_sc_pool():
    @functools.partial(
        pl.kernel,
        out_type=jax.ShapeDtypeStruct((ROWS, CH), jnp.float32),
        mesh=plsc.VectorSubcoreMesh(core_axis_name="c", subcore_axis_name="s"),
        scratch_types=[
            pltpu.VMEM((CHUNK * 4,), jnp.int32),
            pltpu.VMEM((CHUNK * 4,), jnp.float32),
            pltpu.VMEM((CHUNK * 4, CH), jnp.float32),
            pltpu.VMEM((CHUNK, CH), jnp.float32),
            pltpu.SemaphoreType.DMA,
        ],
    )
    def _sc_pool(table_hbm, idx_hbm, w_hbm, out_hbm, idx_v, w_v, rows_v, out_v,
                 sem):
        wid = lax.axis_index("s") * 2 + lax.axis_index("c")

        def chunk_body(ci, carry):
            base = wid * ROWS_PER_W + ci * CHUNK
            pltpu.sync_copy(idx_hbm.at[pl.ds(base * 4, CHUNK * 4)], idx_v)
            pltpu.sync_copy(w_hbm.at[pl.ds(base * 4, CHUNK * 4)], w_v)
            pltpu.async_copy(table_hbm.at[idx_v], rows_v, sem).wait()

            def grp_body(g, gcarry):
                w16 = w_v[pl.ds(16 * g, 16)]
                for rr in range(4):
                    r = 4 * g + rr
                    w0 = w16[4 * rr]
                    w1 = w16[4 * rr + 1]
                    w2 = w16[4 * rr + 2]
                    w3 = w16[4 * rr + 3]
                    for jv in range(CH // 16):
                        s = pl.ds(jv * 16, 16)
                        acc = (rows_v[4 * r, s] * w0 + rows_v[4 * r + 1, s] * w1
                               + rows_v[4 * r + 2, s] * w2
                               + rows_v[4 * r + 3, s] * w3)
                        out_v[r, s] = acc
                return gcarry

            lax.fori_loop(0, CHUNK // 4, grp_body, 0)
            pltpu.sync_copy(out_v, out_hbm.at[pl.ds(base, CHUNK)])
            return carry

        lax.fori_loop(0, NCHUNK, chunk_body, 0)

    return _sc_pool


def kernel(boxes, image_meta, feature_map_p2, feature_map_p3, feature_map_p4,
           feature_map_p5):
    table = jnp.concatenate([
        feature_map_p2.reshape(-1, CH),
        feature_map_p3.reshape(-1, CH),
        feature_map_p4.reshape(-1, CH),
        feature_map_p5.reshape(-1, CH),
    ], axis=0)
    boxes_flat = boxes.reshape(NBOX, 4)
    pad = jnp.broadcast_to(jnp.array([0.0, 0.0, 0.5, 0.5], jnp.float32),
                           (NBOX_PAD - NBOX, 4))
    boxes_pad = jnp.concatenate([boxes_flat, pad], axis=0)
    idx, wgt = _prep(boxes_pad, image_meta)
    out = _make_sc_pool()(table, idx.reshape(-1), wgt.reshape(-1))
    return out[:NBOX * PH * PW].reshape(BB, NN, PH, PW, CH)


# speedup vs baseline: 17.7950x; 1.3884x over previous
import functools

import jax
import jax.numpy as jnp
from jax import lax
from jax.experimental import pallas as pl
from jax.experimental.pallas import tpu as pltpu
from jax.experimental.pallas import tpu_sc as plsc

BB, NN = 2, 1000
PH, PW = 7, 7
CH = 256
NBOX = BB * NN
NBOX_PAD = 2048
ROWS = NBOX_PAD * PH * PW
NWORK = 32
ROWS_PER_W = ROWS // NWORK
CHUNK = 32
NCHUNK = ROWS_PER_W // CHUNK

_HL = (256, 128, 64, 32)
_OFF = (0,
        BB * 256 * 256,
        BB * 256 * 256 + BB * 128 * 128,
        BB * 256 * 256 + BB * 128 * 128 + BB * 64 * 64)
_TROWS = _OFF[3] + BB * 32 * 32


def _prep_body(boxes_ref, meta_ref, idx_ref, w_ref):
    b4 = boxes_ref[:]
    y1 = b4[:, 0:1]
    x1 = b4[:, 1:2]
    y2 = b4[:, 2:3]
    x2 = b4[:, 3:4]
    h = y2 - y1
    w = x2 - x1
    ih = meta_ref[0:1, 4:5]
    iw = meta_ref[0:1, 5:6]
    area = ih * iw
    lvl_f = jnp.log(jnp.sqrt(h * w) / (224.0 / jnp.sqrt(area))) / jnp.log(2.0)
    lvl = jnp.minimum(5, jnp.maximum(2, 4 + jnp.round(lvl_f).astype(jnp.int32)))

    hf = jnp.where(lvl == 2, 256.0,
                   jnp.where(lvl == 3, 128.0,
                             jnp.where(lvl == 4, 64.0, 32.0)))
    hi = hf.astype(jnp.int32)
    hw_rows = hi * hi
    off = jnp.where(lvl == 2, _OFF[0],
                    jnp.where(lvl == 3, _OFF[1],
                              jnp.where(lvl == 4, _OFF[2], _OFF[3])))
    bidx = (lax.broadcasted_iota(jnp.int32, (NBOX_PAD, 1), 0) >= NN).astype(jnp.int32)

    q = lax.broadcasted_iota(jnp.int32, (NBOX_PAD, PH * PW * 4), 1)
    pi = q // (PW * 4)
    pj = (q // 4) % PW
    c = q % 4
    fy = pi.astype(jnp.float32)
    fx = pj.astype(jnp.float32)
    ys = y1 * (hf - 1.0) + fy * (h * (hf - 1.0) / 6.0)
    xs = x1 * (hf - 1.0) + fx * (w * (hf - 1.0) / 6.0)
    y0f = jnp.floor(ys)
    x0f = jnp.floor(xs)
    wy = ys - y0f
    wx = xs - x0f
    him1 = hi - 1
    yy0 = jnp.clip(y0f.astype(jnp.int32), 0, him1)
    yy1 = jnp.minimum(yy0 + 1, him1)
    xx0 = jnp.clip(x0f.astype(jnp.int32), 0, him1)
    xx1 = jnp.minimum(xx0 + 1, him1)
    cy = jnp.where(c >= 2, yy1, yy0)
    cx = jnp.where(c % 2 == 1, xx1, xx0)
    wyc = jnp.where(c >= 2, wy, 1.0 - wy)
    wxc = jnp.where(c % 2 == 1, wx, 1.0 - wx)
    flat = off + bidx * hw_rows + cy * hi + cx
    wgt = wyc * wxc
    valid = ((ys >= 0.0) & (ys <= hf - 1.0) & (xs >= 0.0) & (xs <= hf - 1.0))
    wgt = jnp.where(valid, wgt, 0.0)
    idx_ref[:] = flat
    w_ref[:] = wgt


_prep = pl.pallas_call(
    _prep_body,
    out_shape=[
        jax.ShapeDtypeStruct((NBOX_PAD, PH * PW * 4), jnp.int32),
        jax.ShapeDtypeStruct((NBOX_PAD, PH * PW * 4), jnp.float32),
    ],
)


@functools.cache
def _make_sc_pool():
    @functools.partial(
        pl.kernel,
        out_type=jax.ShapeDtypeStruct((ROWS, CH), jnp.float32),
        mesh=plsc.VectorSubcoreMesh(core_axis_name="c", subcore_axis_name="s"),
        scratch_types=[
            pltpu.VMEM((NCHUNK * CHUNK * 4,), jnp.int32),
            pltpu.VMEM((NCHUNK * CHUNK * 4,), jnp.float32),
            pltpu.VMEM((CHUNK * 4, CH), jnp.float32),
            pltpu.VMEM((CHUNK * 4, CH), jnp.float32),
            pltpu.VMEM((CHUNK, CH), jnp.float32),
            pltpu.VMEM((CHUNK, CH), jnp.float32),
            pltpu.SemaphoreType.DMA,
            pltpu.SemaphoreType.DMA,
            pltpu.SemaphoreType.DMA,
            pltpu.SemaphoreType.DMA,
        ],
    )
    def _sc_pool(table_hbm, idx_hbm, w_hbm, out_hbm, idx_all, w_all,
                 rows_a, rows_b, out_a, out_b, gs_a, gs_b, os_a, os_b):
        wid = lax.axis_index("s") * 2 + lax.axis_index("c")
        rows_bufs = (rows_a, rows_b)
        out_bufs = (out_a, out_b)
        g_sems = (gs_a, gs_b)
        o_sems = (os_a, os_b)

        pltpu.sync_copy(idx_hbm.at[wid], idx_all)
        pltpu.sync_copy(w_hbm.at[wid], w_all)

        def start_gather(ci, b):
            pltpu.async_copy(
                table_hbm.at[idx_all.at[pl.ds(ci * CHUNK * 4, CHUNK * 4)]],
                rows_bufs[b], g_sems[b])

        def compute(ci, b):
            rows_v = rows_bufs[b]
            out_v = out_bufs[b]

            def grp_body(g, gcarry):
                w16 = w_all[pl.ds(ci * CHUNK * 4 + 16 * g, 16)]
                for rr in range(4):
                    r = 4 * g + rr
                    w0 = w16[4 * rr]
                    w1 = w16[4 * rr + 1]
                    w2 = w16[4 * rr + 2]
                    w3 = w16[4 * rr + 3]
                    for jv in range(CH // 16):
                        s = pl.ds(jv * 16, 16)
                        acc = (rows_v[4 * r, s] * w0 + rows_v[4 * r + 1, s] * w1
                               + rows_v[4 * r + 2, s] * w2
                               + rows_v[4 * r + 3, s] * w3)
                        out_v[r, s] = acc
                return gcarry

            lax.fori_loop(0, CHUNK // 4, grp_body, 0)

        start_gather(0, 0)

        def pair_body(ci0, carry):
            for b in range(2):
                ci = 2 * ci0 + b
                nb = 1 - b

                @pl.when(ci + 1 < NCHUNK)
                def _():
                    start_gather(ci + 1, nb)

                pltpu.make_async_copy(
                    table_hbm.at[idx_all.at[pl.ds(ci * CHUNK * 4, CHUNK * 4)]],
                    rows_bufs[b], g_sems[b]).wait()

                @pl.when(ci >= 2)
                def _():
                    pltpu.make_async_copy(
                        out_bufs[b], out_hbm.at[pl.ds(0, CHUNK)],
                        o_sems[b]).wait()

                compute(ci, b)
                base = wid * ROWS_PER_W + ci * CHUNK
                pltpu.async_copy(out_bufs[b], out_hbm.at[pl.ds(base, CHUNK)],
                                 o_sems[b])
            return carry

        lax.fori_loop(0, NCHUNK // 2, pair_body, 0)
        for b in range(2):
            pltpu.make_async_copy(out_bufs[b], out_hbm.at[pl.ds(0, CHUNK)],
                                  o_sems[b]).wait()

    return _sc_pool


def kernel(boxes, image_meta, feature_map_p2, feature_map_p3, feature_map_p4,
           feature_map_p5):
    table = jnp.concatenate([
        feature_map_p2.reshape(-1, CH),
        feature_map_p3.reshape(-1, CH),
        feature_map_p4.reshape(-1, CH),
        feature_map_p5.reshape(-1, CH),
    ], axis=0)
    boxes_flat = boxes.reshape(NBOX, 4)
    pad = jnp.broadcast_to(jnp.array([0.0, 0.0, 0.5, 0.5], jnp.float32),
                           (NBOX_PAD - NBOX, 4))
    boxes_pad = jnp.concatenate([boxes_flat, pad], axis=0)
    idx, wgt = _prep(boxes_pad, image_meta)
    out = _make_sc_pool()(table,
                          idx.reshape(NWORK, NCHUNK * CHUNK * 4),
                          wgt.reshape(NWORK, NCHUNK * CHUNK * 4))
    return out[:NBOX * PH * PW].reshape(BB, NN, PH, PW, CH)
